# baseline (speedup 1.0000x reference)
import functools

import jax
import jax.numpy as jnp
from jax import lax
from jax.experimental import pallas as pl
from jax.experimental.pallas import tpu as pltpu
from jax.experimental.pallas import tpu_sc as plsc

TOPK_K = 128
N_ROWS = 24576
ROW_LEN = 576
LANES = 16
NVEC = ROW_LEN // LANES
N_WORKERS = 32
ROWS_PER_W = N_ROWS // N_WORKERS
CHUNK = 128
N_CHUNKS = ROWS_PER_W // CHUNK
INT_MIN32 = -(2**31)


def _sc_body(x_hbm, o_hbm, buf):
    c = lax.axis_index("c")
    s = lax.axis_index("s")
    wid = s * 2 + c
    row0 = wid * ROWS_PER_W

    def chunk_body(ci, carry):
        base = row0 + ci * CHUNK
        pltpu.sync_copy(x_hbm.at[pl.ds(base, CHUNK)], buf)

        def row_body(r, rcarry):
            keys = []
            for j in range(NVEC):
                b = buf[r, pl.ds(j * LANES, LANES)]
                skey = b ^ ((b >> 31) & jnp.int32(0x7FFFFFFF))
                keys.append(skey)

            def bit_body(i, tb):
                cand_b = tb | jnp.full((LANES,), 1, jnp.int32) << (31 - i)
                cand = cand_b ^ INT_MIN32
                total = jnp.zeros((LANES,), jnp.int32)
                for kj in keys:
                    total = total + plsc.all_reduce_population_count(
                        kj >= cand
                    )
                return jnp.where(total >= TOPK_K, cand_b, tb)

            tb0 = jnp.zeros((LANES,), jnp.int32)
            tb = lax.fori_loop(0, 32, bit_body, tb0)
            t = tb ^ INT_MIN32
            zero = jnp.zeros((LANES,), jnp.int32)
            for j in range(NVEC):
                bv = buf[r, pl.ds(j * LANES, LANES)]
                buf[r, pl.ds(j * LANES, LANES)] = jnp.where(
                    keys[j] >= t, bv, zero
                )
            return rcarry

        lax.fori_loop(0, CHUNK, row_body, 0)
        pltpu.sync_copy(buf, o_hbm.at[pl.ds(base, CHUNK)])
        return carry

    lax.fori_loop(0, N_CHUNKS, chunk_body, 0)


@jax.jit
def _sc_sparsify(xr):
    mesh = plsc.VectorSubcoreMesh(core_axis_name="c", subcore_axis_name="s")
    fn = pl.kernel(
        _sc_body,
        out_type=jax.ShapeDtypeStruct((N_ROWS, ROW_LEN), jnp.int32),
        mesh=mesh,
        compiler_params=pltpu.CompilerParams(needs_layout_passes=False),
        scratch_types=[pltpu.VMEM((CHUNK, ROW_LEN), jnp.int32)],
    )
    return fn(xr)


def kernel(x):
    n, c, h, w = x.shape
    xr = lax.bitcast_convert_type(x.reshape(n * c, h * w), jnp.int32)
    out = _sc_sparsify(xr)
    return lax.bitcast_convert_type(out, jnp.float32).reshape(n, c, h, w)
